# Initial kernel scaffold; baseline (speedup 1.0000x reference)
import jax
import jax.numpy as jnp
from jax.experimental import pallas as pl


def kernel(x, edge_index, edge_attr, h, W_z, b_z, W_r, b_r, W_h, b_h, lin_w, lin_b):
    raise NotImplementedError("write your pallas kernel here")


# speedup vs baseline: 1.4736x; 1.4736x over previous
import jax
import jax.numpy as jnp
from jax.experimental import pallas as pl
from jax.experimental.pallas import tpu as pltpu


def _cell_kernel(x_ref, h_ref, wx_ref, wh_ref, whh_ref, bias_ref, lin_ref,
                 linb_ref, out_ref, H_ref):
    x = x_ref[...]
    h = h_ref[...]
    wx = wx_ref[...]
    wh = wh_ref[...]
    whh = whh_ref[...]
    bias = bias_ref[...]
    lin = lin_ref[...]
    linb = linb_ref[...]

    D = h.shape[1]
    gx = jnp.dot(x, wx, preferred_element_type=jnp.float32)
    gh = jnp.dot(h, wh, preferred_element_type=jnp.float32)

    z = jax.nn.sigmoid(gx[:, :D] + gh[:, :D] + bias[:, :D])
    r = jax.nn.sigmoid(gx[:, D:2 * D] + gh[:, D:2 * D] + bias[:, D:2 * D])
    ht = jnp.tanh(gx[:, 2 * D:] +
                  jnp.dot(r * h, whh, preferred_element_type=jnp.float32) +
                  bias[:, 2 * D:])
    H = z * h + (1.0 - z) * ht
    H_ref[...] = H
    out_ref[...] = (jnp.sum(jnp.maximum(H, 0.0) * lin, axis=1, keepdims=True)
                    + linb)


def kernel(x, edge_index, edge_attr, h, W_z, b_z, W_r, b_r, W_h, b_h,
           lin_w, lin_b):
    del edge_index, edge_attr
    N, F = x.shape
    D = h.shape[1]

    Wz = W_z[0, 0] + W_z[1, 0]
    Wr = W_r[0, 0] + W_r[1, 0]
    Wh = W_h[0, 0] + W_h[1, 0]
    wx = jnp.concatenate([Wz[:F], Wr[:F], Wh[:F]], axis=1)
    wh = jnp.concatenate([Wz[F:], Wr[F:]], axis=1)
    whh = Wh[F:]
    bias = jnp.concatenate([b_z, b_r, b_h]).reshape(1, 3 * D)
    lin = lin_w.reshape(1, D)
    linb = lin_b.reshape(1, 1)

    B = 2000
    grid = (N // B,)

    out, H = pl.pallas_call(
        _cell_kernel,
        grid=grid,
        in_specs=[
            pl.BlockSpec((B, F), lambda i: (i, 0)),
            pl.BlockSpec((B, D), lambda i: (i, 0)),
            pl.BlockSpec((F, 3 * D), lambda i: (0, 0)),
            pl.BlockSpec((D, 2 * D), lambda i: (0, 0)),
            pl.BlockSpec((D, D), lambda i: (0, 0)),
            pl.BlockSpec((1, 3 * D), lambda i: (0, 0)),
            pl.BlockSpec((1, D), lambda i: (0, 0)),
            pl.BlockSpec((1, 1), lambda i: (0, 0)),
        ],
        out_specs=[
            pl.BlockSpec((B, 1), lambda i: (i, 0)),
            pl.BlockSpec((B, D), lambda i: (i, 0)),
        ],
        out_shape=[
            jax.ShapeDtypeStruct((N, 1), jnp.float32),
            jax.ShapeDtypeStruct((N, D), jnp.float32),
        ],
        compiler_params=pltpu.CompilerParams(
            dimension_semantics=("arbitrary",),
        ),
    )(x, h, wx, wh, whh, bias, lin, linb)
    return (out, H)
